# baseline (speedup 1.0000x reference)
import functools

import jax
import jax.numpy as jnp
from jax import lax
from jax.experimental import pallas as pl
from jax.experimental.pallas import tpu as pltpu
from jax.experimental.pallas import tpu_sc as plsc

B = 4096
N_TOK = 26
TOK_DIM = 10000
N_FLOAT = 13
SEQ_VOCAB = 100000
HIST = 50
D = 64

NC = 2
NSUB = 16
NW = NC * NSUB
PB = B // NW
NB = 4
NSTEPS = PB // NB
HISTP = 64
BC = 512
TBC = 32768

_mesh = plsc.VectorSubcoreMesh(core_axis_name="c", subcore_axis_name="s")


def _tr_body(i_ref, o_ref):
    a = i_ref[...].T
    o_ref[...] = jnp.concatenate(
        [jnp.concatenate([a[g * BC: g * BC + BC // 2],
                          a[g * BC + BC // 2: (g + 1) * BC]], axis=1)
         for g in range(TBC // BC)], axis=0)


def _transpose_table(table_t, v):
    grid = (v + TBC - 1) // TBC
    out = pl.pallas_call(
        _tr_body,
        grid=(grid,),
        in_specs=[pl.BlockSpec((D, TBC), lambda c: (0, c))],
        out_specs=pl.BlockSpec((TBC // 2, 128), lambda c: (c, 0)),
        out_shape=jax.ShapeDtypeStruct((grid * (TBC // 2), 128), jnp.float32),
    )(table_t)
    return out.reshape(grid * TBC, D)


def _remap(v):
    return (v & -512) + ((v & 255) << 1) + ((v >> 8) & 1)


def _dense_body(ft_ref, ff_ref, o_ref):
    ft = ft_ref[...]
    ff = ff_ref[...]
    o_ref[...] = ft[:, :, None] * ff[:, None, :]


_dense_call = pl.pallas_call(
    _dense_body,
    grid=(B // 512,),
    in_specs=[pl.BlockSpec((N_FLOAT, D), lambda i: (0, 0)),
              pl.BlockSpec((N_FLOAT, 512), lambda i: (0, i))],
    out_specs=pl.BlockSpec((N_FLOAT, D, 512), lambda i: (0, 0, i)),
    out_shape=jax.ShapeDtypeStruct((N_FLOAT, D, B), jnp.float32),
)


@functools.partial(
    pl.kernel,
    out_type=jax.ShapeDtypeStruct((B, D), jnp.float32),
    mesh=_mesh,
    scratch_types=(
        pltpu.VMEM((PB * HISTP,), jnp.int32),
        pltpu.VMEM((1, D), jnp.float32),
        pltpu.VMEM((NB * HIST, D), jnp.float32),
        pltpu.VMEM((NB * HIST, D), jnp.float32),
        pltpu.VMEM((NB, D), jnp.float32),
        pltpu.VMEM((NB, D), jnp.float32),
        pltpu.SemaphoreType.DMA,
        pltpu.SemaphoreType.DMA,
        pltpu.SemaphoreType.DMA,
        pltpu.SemaphoreType.DMA,
    ),
    compiler_params=pltpu.CompilerParams(use_tc_tiling_on_sc=False,
                                         needs_layout_passes=False),
)
def _sc_pool(seq_idx, seq_table,
             out_pool,
             idx_seq_v, row0_v, seq0, seq1, pool0, pool1,
             sem_g0, sem_g1, sem_o0, sem_o1):
    wid = lax.axis_index("s") * NC + lax.axis_index("c")

    pltpu.sync_copy(seq_idx.at[pl.ds(wid * (PB * HISTP), PB * HISTP)], idx_seq_v)
    pltpu.sync_copy(seq_table.at[pl.ds(0, 1)], row0_v)

    lane = lax.iota(jnp.int32, 16)
    tail_mask = lane < 2

    def issue(s, seqbuf, sem):
        g0 = s * NB
        for j in range(NB):
            pltpu.async_copy(
                seq_table.at[idx_seq_v.at[pl.ds((g0 + j) * HISTP, HIST)]],
                seqbuf.at[pl.ds(j * HIST, HIST)], sem)

    def process(s, seqbuf, poolbuf, sem_g, sem_o, t):
        g0 = s * NB
        b0 = wid * PB + g0

        pltpu.make_async_copy(
            seq_table.at[pl.ds(0, NB * HIST)], seqbuf, sem_g).wait()

        @pl.when(t > 0)
        def _():
            pltpu.make_async_copy(
                seq_table.at[pl.ds(0, NB)], poolbuf, sem_o).wait()

        for j in range(NB):
            g = g0 + j
            off = g * HISTP
            k0 = idx_seq_v[pl.ds(off, 16)]
            k1 = idx_seq_v[pl.ds(off + 16, 16)]
            k2 = idx_seq_v[pl.ds(off + 32, 16)]
            k3 = idx_seq_v[pl.ds(off + 48, 16)]
            nz = (plsc.all_reduce_population_count(k0 == 0)
                  + plsc.all_reduce_population_count(k1 == 0)
                  + plsc.all_reduce_population_count(k2 == 0)
                  + plsc.all_reduce_population_count((k3 == 0) & tail_mask))
            zf = nz.astype(jnp.float32)
            cnt = 50.0 - zf

            def body(h, accs):
                r = j * HIST + h
                return tuple(
                    accs[q] + seqbuf[r, pl.ds(q * 16, 16)] for q in range(4))

            zero = jnp.zeros((16,), jnp.float32)
            accs = lax.fori_loop(0, HIST, body, (zero, zero, zero, zero))
            for q in range(4):
                p = (accs[q] - zf * row0_v[0, pl.ds(q * 16, 16)]) / (cnt + 1e-8)
                poolbuf[j, pl.ds(q * 16, 16)] = jnp.where(cnt > 0.0, p, 0.0)

        pltpu.async_copy(poolbuf, out_pool.at[pl.ds(b0, NB)], sem_o)

        @pl.when(t < NSTEPS // 2 - 1)
        def _():
            issue(s + 2, seqbuf, sem_g)

    issue(0, seq0, sem_g0)
    issue(1, seq1, sem_g1)

    def pair(t, _):
        process(2 * t, seq0, pool0, sem_g0, sem_o0, t)
        process(2 * t + 1, seq1, pool1, sem_g1, sem_o1, t)
        return 0

    lax.fori_loop(0, NSTEPS // 2, pair, 0)
    pltpu.make_async_copy(seq_table.at[pl.ds(0, NB)], pool0, sem_o0).wait()
    pltpu.make_async_copy(seq_table.at[pl.ds(0, NB)], pool1, sem_o1).wait()


@functools.partial(
    pl.kernel,
    out_type=jax.ShapeDtypeStruct((B * (N_TOK + 1), D), jnp.float32),
    mesh=_mesh,
    scratch_types=(
        pltpu.VMEM((PB * N_TOK,), jnp.int32),
        pltpu.VMEM((PB, D), jnp.float32),
        pltpu.VMEM((NB * N_TOK, D), jnp.float32),
        pltpu.VMEM((NB * N_TOK, D), jnp.float32),
        pltpu.SemaphoreType.DMA,
        pltpu.SemaphoreType.DMA,
        pltpu.SemaphoreType.DMA,
        pltpu.SemaphoreType.DMA,
    ),
    compiler_params=pltpu.CompilerParams(use_tc_tiling_on_sc=False,
                                         needs_layout_passes=False),
)
def _sc_tok(tok_idx, pooled, tok_table,
            out_sp,
            idx_tok_v, pool_v, tok0, tok1,
            sem_g0, sem_g1, sem_o0, sem_o1):
    wid = lax.axis_index("s") * NC + lax.axis_index("c")

    pltpu.sync_copy(tok_idx.at[pl.ds(wid * (PB * N_TOK), PB * N_TOK)], idx_tok_v)
    pltpu.sync_copy(pooled.at[pl.ds(wid * PB, PB)], pool_v)

    def issue(s, tokbuf, sem):
        g0 = s * NB
        pltpu.async_copy(
            tok_table.at[idx_tok_v.at[pl.ds(g0 * N_TOK, NB * N_TOK)]],
            tokbuf, sem)

    def process(s, tokbuf, sem_g, sem_o, t):
        g0 = s * NB
        orow = (wid * PB + g0) * (N_TOK + 1)

        pltpu.make_async_copy(
            tok_table.at[pl.ds(0, NB * N_TOK)], tokbuf, sem_g).wait()

        outs = []
        for j in range(NB):
            outs.append(pltpu.async_copy(
                tokbuf.at[pl.ds(j * N_TOK, N_TOK)],
                out_sp.at[pl.ds(orow + j * (N_TOK + 1), N_TOK)], sem_o))
            outs.append(pltpu.async_copy(
                pool_v.at[pl.ds(g0 + j, 1)],
                out_sp.at[pl.ds(orow + j * (N_TOK + 1) + N_TOK, 1)], sem_o))
        for o in outs:
            o.wait()

        @pl.when(t < NSTEPS // 2 - 1)
        def _():
            issue(s + 2, tokbuf, sem_g)

    issue(0, tok0, sem_g0)
    issue(1, tok1, sem_g1)

    def pair(t, _):
        process(2 * t, tok0, sem_g0, sem_o0, t)
        process(2 * t + 1, tok1, sem_g1, sem_o1, t)
        return 0

    lax.fori_loop(0, NSTEPS // 2, pair, 0)


def kernel(token_fields, float_fields, token_seq_field, token_table,
           float_table, seq_table):
    offsets = (jnp.arange(N_TOK, dtype=jnp.int32) * TOK_DIM)[None, :]
    tok_idx = _remap(token_fields.astype(jnp.int32) + offsets).reshape(-1)
    seq_idx = jnp.pad(_remap(token_seq_field.astype(jnp.int32)),
                      ((0, 0), (0, HISTP - HIST)),
                      constant_values=1).reshape(-1)

    seq_lin = _transpose_table(seq_table.T, SEQ_VOCAB)
    pooled = _sc_pool(seq_idx, seq_lin)
    tok_lin = _transpose_table(token_table.T, TOK_DIM * N_TOK)
    out_sp = _sc_tok(tok_idx, pooled, tok_lin)
    dense = _dense_call(float_table, float_fields.T)

    return (out_sp.reshape(B, N_TOK + 1, D), jnp.transpose(dense, (2, 0, 1)))


# speedup vs baseline: 1.0255x; 1.0255x over previous
import functools

import jax
import jax.numpy as jnp
from jax import lax
from jax.experimental import pallas as pl
from jax.experimental.pallas import tpu as pltpu
from jax.experimental.pallas import tpu_sc as plsc

B = 4096
N_TOK = 26
TOK_DIM = 10000
N_FLOAT = 13
SEQ_VOCAB = 100000
HIST = 50
D = 64

NC = 2
NSUB = 16
NW = NC * NSUB
PB = B // NW
NB = 4
NSTEPS = PB // NB
HISTP = 64
BC = 512
TBC = 16384

_mesh = plsc.VectorSubcoreMesh(core_axis_name="c", subcore_axis_name="s")


def _tr_body(i_ref, o_ref):
    a = i_ref[...].T
    o_ref[...] = jnp.concatenate(
        [jnp.concatenate([a[g * BC: g * BC + BC // 2],
                          a[g * BC + BC // 2: (g + 1) * BC]], axis=1)
         for g in range(TBC // BC)], axis=0)


def _transpose_table(table_t, v):
    grid = (v + TBC - 1) // TBC
    out = pl.pallas_call(
        _tr_body,
        grid=(grid,),
        in_specs=[pl.BlockSpec((D, TBC), lambda c: (0, c))],
        out_specs=pl.BlockSpec((TBC // 2, 128), lambda c: (c, 0)),
        out_shape=jax.ShapeDtypeStruct((grid * (TBC // 2), 128), jnp.float32),
    )(table_t)
    return out.reshape(grid * TBC, D)


def _remap(v):
    return (v & -512) + ((v & 255) << 1) + ((v >> 8) & 1)


def _dense_body(ft_ref, ff_ref, o_ref):
    ft = ft_ref[...]
    ff = ff_ref[...]
    o_ref[...] = ft[:, :, None] * ff[:, None, :]


_dense_call = pl.pallas_call(
    _dense_body,
    grid=(B // 512,),
    in_specs=[pl.BlockSpec((N_FLOAT, D), lambda i: (0, 0)),
              pl.BlockSpec((N_FLOAT, 512), lambda i: (0, i))],
    out_specs=pl.BlockSpec((N_FLOAT, D, 512), lambda i: (0, 0, i)),
    out_shape=jax.ShapeDtypeStruct((N_FLOAT, D, B), jnp.float32),
)


@functools.partial(
    pl.kernel,
    out_type=jax.ShapeDtypeStruct((B, D), jnp.float32),
    mesh=_mesh,
    scratch_types=(
        pltpu.VMEM((PB * HISTP,), jnp.int32),
        pltpu.VMEM((1, D), jnp.float32),
        pltpu.VMEM((NB * HIST, D), jnp.float32),
        pltpu.VMEM((NB * HIST, D), jnp.float32),
        pltpu.VMEM((NB, D), jnp.float32),
        pltpu.VMEM((NB, D), jnp.float32),
        pltpu.SemaphoreType.DMA,
        pltpu.SemaphoreType.DMA,
        pltpu.SemaphoreType.DMA,
        pltpu.SemaphoreType.DMA,
    ),
    compiler_params=pltpu.CompilerParams(use_tc_tiling_on_sc=False,
                                         needs_layout_passes=False),
)
def _sc_pool(seq_idx, seq_table,
             out_pool,
             idx_seq_v, row0_v, seq0, seq1, pool0, pool1,
             sem_g0, sem_g1, sem_o0, sem_o1):
    wid = lax.axis_index("s") * NC + lax.axis_index("c")

    pltpu.sync_copy(seq_idx.at[pl.ds(wid * (PB * HISTP), PB * HISTP)], idx_seq_v)
    pltpu.sync_copy(seq_table.at[pl.ds(0, 1)], row0_v)

    lane = lax.iota(jnp.int32, 16)
    tail_mask = lane < 2

    def issue(s, seqbuf, sem):
        g0 = s * NB
        for j in range(NB):
            pltpu.async_copy(
                seq_table.at[idx_seq_v.at[pl.ds((g0 + j) * HISTP, HIST)]],
                seqbuf.at[pl.ds(j * HIST, HIST)], sem)

    def process(s, seqbuf, poolbuf, sem_g, sem_o, t):
        g0 = s * NB
        b0 = wid * PB + g0

        pltpu.make_async_copy(
            seq_table.at[pl.ds(0, NB * HIST)], seqbuf, sem_g).wait()

        @pl.when(t > 0)
        def _():
            pltpu.make_async_copy(
                seq_table.at[pl.ds(0, NB)], poolbuf, sem_o).wait()

        for j in range(NB):
            g = g0 + j
            off = g * HISTP
            k0 = idx_seq_v[pl.ds(off, 16)]
            k1 = idx_seq_v[pl.ds(off + 16, 16)]
            k2 = idx_seq_v[pl.ds(off + 32, 16)]
            k3 = idx_seq_v[pl.ds(off + 48, 16)]
            nz = (plsc.all_reduce_population_count(k0 == 0)
                  + plsc.all_reduce_population_count(k1 == 0)
                  + plsc.all_reduce_population_count(k2 == 0)
                  + plsc.all_reduce_population_count((k3 == 0) & tail_mask))
            zf = nz.astype(jnp.float32)
            cnt = 50.0 - zf

            def body(h, accs):
                r = j * HIST + h
                return tuple(
                    accs[q] + seqbuf[r, pl.ds(q * 16, 16)] for q in range(4))

            zero = jnp.zeros((16,), jnp.float32)
            accs = lax.fori_loop(0, HIST, body, (zero, zero, zero, zero))
            for q in range(4):
                p = (accs[q] - zf * row0_v[0, pl.ds(q * 16, 16)]) / (cnt + 1e-8)
                poolbuf[j, pl.ds(q * 16, 16)] = jnp.where(cnt > 0.0, p, 0.0)

        pltpu.async_copy(poolbuf, out_pool.at[pl.ds(b0, NB)], sem_o)

        @pl.when(t < NSTEPS // 2 - 1)
        def _():
            issue(s + 2, seqbuf, sem_g)

    issue(0, seq0, sem_g0)
    issue(1, seq1, sem_g1)

    def pair(t, _):
        process(2 * t, seq0, pool0, sem_g0, sem_o0, t)
        process(2 * t + 1, seq1, pool1, sem_g1, sem_o1, t)
        return 0

    lax.fori_loop(0, NSTEPS // 2, pair, 0)
    pltpu.make_async_copy(seq_table.at[pl.ds(0, NB)], pool0, sem_o0).wait()
    pltpu.make_async_copy(seq_table.at[pl.ds(0, NB)], pool1, sem_o1).wait()


@functools.partial(
    pl.kernel,
    out_type=jax.ShapeDtypeStruct((B * (N_TOK + 1), D), jnp.float32),
    mesh=_mesh,
    scratch_types=(
        pltpu.VMEM((PB * N_TOK,), jnp.int32),
        pltpu.VMEM((PB, D), jnp.float32),
        pltpu.VMEM((NB * N_TOK, D), jnp.float32),
        pltpu.VMEM((NB * N_TOK, D), jnp.float32),
        pltpu.SemaphoreType.DMA,
        pltpu.SemaphoreType.DMA,
        pltpu.SemaphoreType.DMA,
        pltpu.SemaphoreType.DMA,
    ),
    compiler_params=pltpu.CompilerParams(use_tc_tiling_on_sc=False,
                                         needs_layout_passes=False),
)
def _sc_tok(tok_idx, pooled, tok_table,
            out_sp,
            idx_tok_v, pool_v, tok0, tok1,
            sem_g0, sem_g1, sem_o0, sem_o1):
    wid = lax.axis_index("s") * NC + lax.axis_index("c")

    pltpu.sync_copy(tok_idx.at[pl.ds(wid * (PB * N_TOK), PB * N_TOK)], idx_tok_v)
    pltpu.sync_copy(pooled.at[pl.ds(wid * PB, PB)], pool_v)

    def issue(s, tokbuf, sem):
        g0 = s * NB
        pltpu.async_copy(
            tok_table.at[idx_tok_v.at[pl.ds(g0 * N_TOK, NB * N_TOK)]],
            tokbuf, sem)

    def process(s, tokbuf, sem_g, sem_o, t):
        g0 = s * NB
        orow = (wid * PB + g0) * (N_TOK + 1)

        pltpu.make_async_copy(
            tok_table.at[pl.ds(0, NB * N_TOK)], tokbuf, sem_g).wait()

        outs = []
        for j in range(NB):
            outs.append(pltpu.async_copy(
                tokbuf.at[pl.ds(j * N_TOK, N_TOK)],
                out_sp.at[pl.ds(orow + j * (N_TOK + 1), N_TOK)], sem_o))
            outs.append(pltpu.async_copy(
                pool_v.at[pl.ds(g0 + j, 1)],
                out_sp.at[pl.ds(orow + j * (N_TOK + 1) + N_TOK, 1)], sem_o))
        for o in outs:
            o.wait()

        @pl.when(t < NSTEPS // 2 - 1)
        def _():
            issue(s + 2, tokbuf, sem_g)

    issue(0, tok0, sem_g0)
    issue(1, tok1, sem_g1)

    def pair(t, _):
        process(2 * t, tok0, sem_g0, sem_o0, t)
        process(2 * t + 1, tok1, sem_g1, sem_o1, t)
        return 0

    lax.fori_loop(0, NSTEPS // 2, pair, 0)


def kernel(token_fields, float_fields, token_seq_field, token_table,
           float_table, seq_table):
    offsets = (jnp.arange(N_TOK, dtype=jnp.int32) * TOK_DIM)[None, :]
    tok_idx = _remap(token_fields.astype(jnp.int32) + offsets).reshape(-1)
    seq_idx = jnp.pad(_remap(token_seq_field.astype(jnp.int32)),
                      ((0, 0), (0, HISTP - HIST)),
                      constant_values=1).reshape(-1)

    seq_lin = _transpose_table(seq_table.T, SEQ_VOCAB)
    pooled = _sc_pool(seq_idx, seq_lin)
    tok_lin = _transpose_table(token_table.T, TOK_DIM * N_TOK)
    out_sp = _sc_tok(tok_idx, pooled, tok_lin)
    dense = _dense_call(float_table, float_fields.T)

    return (out_sp.reshape(B, N_TOK + 1, D), jnp.transpose(dense, (2, 0, 1)))
